# baseline (speedup 1.0000x reference)
import functools

import jax
import jax.numpy as jnp
from jax import lax
from jax.experimental import pallas as pl
from jax.experimental.pallas import tpu as pltpu
from jax.experimental.pallas import tpu_sc as plsc

N = 50000
E = 800000
DIN = 25
DIM = 64
FAB = 16
B = 512
H = DIM // 2

NC = 2
NS = 16
L = 16


_TN1 = 1000


def _node_prework_body(x_ref, w0_ref, b0_ref, wm_ref, wroot_ref, g_ref, hr_ref):
    h = jax.nn.relu(jnp.dot(x_ref[...], w0_ref[...],
                            preferred_element_type=jnp.float32) + b0_ref[...])
    g = jnp.dot(h, wm_ref[...], preferred_element_type=jnp.float32)
    g_ref[0] = g[:, :H]
    g_ref[1] = g[:, H:]
    hr_ref[...] = jnp.dot(h, wroot_ref[...], preferred_element_type=jnp.float32)


def _node_prework(x, W0, b0, Wm, Wroot):
    grid = (N // _TN1,)
    return pl.pallas_call(
        _node_prework_body,
        grid=grid,
        in_specs=[
            pl.BlockSpec((_TN1, DIN), lambda i: (i, 0)),
            pl.BlockSpec((DIN, DIM), lambda i: (0, 0)),
            pl.BlockSpec((1, DIM), lambda i: (0, 0)),
            pl.BlockSpec((DIM, DIM), lambda i: (0, 0)),
            pl.BlockSpec((DIM, DIM), lambda i: (0, 0)),
        ],
        out_specs=[
            pl.BlockSpec((2, _TN1, H), lambda i: (0, i, 0)),
            pl.BlockSpec((_TN1, DIM), lambda i: (i, 0)),
        ],
        out_shape=[
            jax.ShapeDtypeStruct((2, N, H), jnp.float32),
            jax.ShapeDtypeStruct((N, DIM), jnp.float32),
        ],
    )(x, W0, b0.reshape(1, DIM), Wm, Wroot)


E_PAD = 802816
EQ = E_PAD // 4
_TE4 = 2048


def _edge_prework_body(a0_ref, a1_ref, a2_ref, a3_ref, we_ref, bc_ref, ea_ref):
    rs = [jnp.dot(a_ref[...], we_ref[...],
                  preferred_element_type=jnp.float32) + bc_ref[...]
          for a_ref in (a0_ref, a1_ref, a2_ref, a3_ref)]
    ea_ref[0] = jnp.concatenate([r[:, :H] for r in rs], axis=1)
    ea_ref[1] = jnp.concatenate([r[:, H:] for r in rs], axis=1)


_NEB = E // _TE4 if E % _TE4 == 0 else E // _TE4 + 1
_GB = EQ // _TE4


def _mk_attr_spec(m):
    return pl.BlockSpec(
        (_TE4, FAB), lambda i, m=m: (jnp.minimum(m * _GB + i, _NEB - 1), 0))


def _edge_prework(edge_attr, We, bconv):
    grid = (_GB,)
    return pl.pallas_call(
        _edge_prework_body,
        grid=grid,
        in_specs=[
            _mk_attr_spec(0),
            _mk_attr_spec(1),
            _mk_attr_spec(2),
            _mk_attr_spec(3),
            pl.BlockSpec((FAB, DIM), lambda i: (0, 0)),
            pl.BlockSpec((1, DIM), lambda i: (0, 0)),
        ],
        out_specs=pl.BlockSpec((2, _TE4, 128), lambda i: (0, i, 0)),
        out_shape=jax.ShapeDtypeStruct((2, EQ, 128), jnp.float32),
    )(edge_attr, edge_attr, edge_attr, edge_attr, We, bconv.reshape(1, DIM))


EPT = E_PAD // NS
CH = 128
ERC = CH // 4
NCHK = EPT // CH
NPAIR = (NCHK - 2) // 2
NPAD = 50176
RPT = NPAD // NS
ZR = 56
DUMMY = N


def _sc_edge_body(g_hbm, ea_hbm, src_hbm, dst_hbm, out_hbm,
                  src0, dst0, ea0, gr0, src1, dst1, ea1, gr1,
                  zbuf, aggr_sh, si0, sg0, ss0, si1, sg1, ss1):
    c = lax.axis_index("c")
    s = lax.axis_index("s")
    zvec = jnp.zeros((L,), jnp.float32)

    def zrow(r, _):
        zbuf[r, pl.ds(0, L)] = zvec
        zbuf[r, pl.ds(L, L)] = zvec
        return 0
    lax.fori_loop(0, ZR, zrow, 0)
    def zcopy(z, _):
        pltpu.sync_copy(zbuf, aggr_sh.at[pl.ds(s * RPT + z * ZR, ZR)])
        return 0
    lax.fori_loop(0, RPT // ZR, zcopy, 0)
    plsc.subcore_barrier()

    ebase = s * EPT
    arow0 = c * EQ + s * (EPT // 4)
    goff = c * N

    bufs = ((src0, dst0, ea0, gr0, si0, sg0, ss0),
            (src1, dst1, ea1, gr1, si1, sg1, ss1))

    def fire_in(j, u):
        sv, dv, ev, gv, si, sg, ss = bufs[u]
        qbase = s * (EPT // 4) + j * ERC
        for m in range(4):
            pltpu.async_copy(src_hbm.at[pl.ds(m * EQ + qbase, ERC)],
                             sv.at[pl.ds(m * ERC, ERC)], si)
            pltpu.async_copy(dst_hbm.at[pl.ds(m * EQ + qbase, ERC)],
                             dv.at[pl.ds(m * ERC, ERC)], si)
        pltpu.async_copy(ea_hbm.at[pl.ds(arow0 + j * ERC, ERC)], ev, si)

    def wait_in(u):
        sv, dv, ev, gv, si, sg, ss = bufs[u]
        for m in range(4):
            pltpu.make_async_copy(src_hbm.at[pl.ds(0, ERC)],
                                  sv.at[pl.ds(m * ERC, ERC)], si).wait()
            pltpu.make_async_copy(dst_hbm.at[pl.ds(0, ERC)],
                                  dv.at[pl.ds(m * ERC, ERC)], si).wait()
        pltpu.make_async_copy(ea_hbm.at[pl.ds(arow0, ERC)], ev, si).wait()

    def add_off(u):
        sv = bufs[u][0]
        for w in range(CH // L):
            sv[pl.ds(w * L, L)] = sv[pl.ds(w * L, L)] + goff

    def fire_gather(u):
        sv, dv, ev, gv, si, sg, ss = bufs[u]
        return pltpu.async_copy(g_hbm.at[sv], gv, sg)

    def compute(u):
        sv, dv, ev, gv, si, sg, ss = bufs[u]
        def body(k, _):
            for m in range(4):
                r = m * ERC + k
                for hh in range(2):
                    w = 2 * m + hh
                    val = gv[r, pl.ds(hh * L, L)] + ev[k, pl.ds(w * L, L)]
                    gv[r, pl.ds(hh * L, L)] = jnp.maximum(val, 0.0)
            return 0
        lax.fori_loop(0, ERC, body, 0)

    def fire_scatter(u):
        sv, dv, ev, gv, si, sg, ss = bufs[u]
        pltpu.async_copy(gv, aggr_sh.at[dv], ss, add=True)

    def wait_scatter(u):
        sv, dv, ev, gv, si, sg, ss = bufs[u]
        pltpu.make_async_copy(gv, aggr_sh.at[dv], ss).wait()

    fire_in(0, 1)
    fire_in(1, 0)
    wait_in(1)
    add_off(1)
    fire_gather(1).wait()
    compute(1)
    fire_scatter(1)

    def do_pair(t, _):
        j0 = 1 + 2 * t
        wait_in(0)
        add_off(0)
        gd0 = fire_gather(0)
        wait_scatter(1)
        fire_in(j0 + 1, 1)
        gd0.wait()
        compute(0)
        fire_scatter(0)
        wait_in(1)
        add_off(1)
        gd1 = fire_gather(1)
        wait_scatter(0)
        fire_in(j0 + 2, 0)
        gd1.wait()
        compute(1)
        fire_scatter(1)
        return 0
    lax.fori_loop(0, NPAIR, do_pair, 0)

    wait_in(0)
    add_off(0)
    fire_gather(0).wait()
    compute(0)
    wait_scatter(1)
    sv, dv, ev, gv, si, sg, ss = bufs[0]
    pltpu.sync_copy(gv, aggr_sh.at[dv], add=True)

    plsc.subcore_barrier()
    pltpu.sync_copy(aggr_sh.at[pl.ds(s * RPT, RPT)],
                    out_hbm.at[pl.ds(c * NPAD + s * RPT, RPT)])


def _sc_edge_aggregate(g_flat, ea_flat, src, dst):
    mesh = plsc.VectorSubcoreMesh(core_axis_name="c", subcore_axis_name="s")
    return pl.kernel(
        _sc_edge_body,
        out_type=jax.ShapeDtypeStruct((2 * NPAD, H), jnp.float32),
        mesh=mesh,
        compiler_params=pltpu.CompilerParams(use_tc_tiling_on_sc=False),
        scratch_types=[
            pltpu.VMEM((CH,), jnp.int32),
            pltpu.VMEM((CH,), jnp.int32),
            pltpu.VMEM((ERC, 128), jnp.float32),
            pltpu.VMEM((CH, H), jnp.float32),
            pltpu.VMEM((CH,), jnp.int32),
            pltpu.VMEM((CH,), jnp.int32),
            pltpu.VMEM((ERC, 128), jnp.float32),
            pltpu.VMEM((CH, H), jnp.float32),
            pltpu.VMEM((ZR, H), jnp.float32),
            pltpu.VMEM_SHARED((NPAD, H), jnp.float32),
            pltpu.SemaphoreType.DMA,
            pltpu.SemaphoreType.DMA,
            pltpu.SemaphoreType.DMA,
            pltpu.SemaphoreType.DMA,
            pltpu.SemaphoreType.DMA,
            pltpu.SemaphoreType.DMA,
        ],
    )(g_flat, ea_flat, src, dst)


def _h2_body(hr_ref, a_ref, h2_ref):
    a = jnp.concatenate([a_ref[0], a_ref[1]], axis=-1)
    h2_ref[...] = jax.nn.relu(hr_ref[...] + a)


def _h2_assemble(hr, aggr):
    grid = (N // _TN1,)
    return pl.pallas_call(
        _h2_body,
        grid=grid,
        in_specs=[
            pl.BlockSpec((_TN1, DIM), lambda i: (i, 0)),
            pl.BlockSpec((2, _TN1, H), lambda i: (0, i, 0)),
        ],
        out_specs=pl.BlockSpec((_TN1, DIM), lambda i: (i, 0)),
        out_shape=jax.ShapeDtypeStruct((N, DIM), jnp.float32),
    )(hr, aggr)


_TN2 = 2000
_NB = N // _TN2
_STEPS = 3
_NEG = -1e30


def _set2set_body(h2_ref, batch_ref, wi_ref, wh_ref, bl_ref,
                  w1_ref, b1_ref, w2_ref, b2_ref, out_ref,
                  hs_scr, cs_scr, q_scr, m_scr, sacc_scr, rnum_scr):
    st = pl.program_id(0)
    j = pl.program_id(1)

    @pl.when(j == 0)
    def lstm_phase():
        is0 = (st == 0).astype(jnp.float32)
        r_t = rnum_scr[...] / (sacc_scr[...] + 1e-16)
        r = r_t.T
        q_prev = q_scr[...]
        keep = 1.0 - is0
        q_star = jnp.concatenate([q_prev, r], axis=-1)
        q_star = jnp.where(st == 0, 0.0, q_star)
        hs_prev = jnp.where(st == 0, 0.0, hs_scr[...])
        cs_prev = jnp.where(st == 0, 0.0, cs_scr[...])
        del keep
        z = (jnp.dot(q_star, wi_ref[...], preferred_element_type=jnp.float32)
             + jnp.dot(hs_prev, wh_ref[...], preferred_element_type=jnp.float32)
             + bl_ref[...])
        i_g = jax.nn.sigmoid(z[:, 0 * DIM:1 * DIM])
        f_g = jax.nn.sigmoid(z[:, 1 * DIM:2 * DIM])
        g_g = jnp.tanh(z[:, 2 * DIM:3 * DIM])
        o_g = jax.nn.sigmoid(z[:, 3 * DIM:4 * DIM])
        cs_new = f_g * cs_prev + i_g * g_g
        hs_new = o_g * jnp.tanh(cs_new)
        hs_scr[...] = hs_new
        cs_scr[...] = cs_new
        q_scr[...] = hs_new
        m_scr[...] = jnp.full((1, B), _NEG, jnp.float32)
        sacc_scr[...] = jnp.zeros((1, B), jnp.float32)
        rnum_scr[...] = jnp.zeros((DIM, B), jnp.float32)

    @pl.when((j >= 1) & (j <= _NB))
    def node_phase():
        h2b = h2_ref[...]
        bb = batch_ref[0, 0, :]
        iota = lax.broadcasted_iota(jnp.int32, (1, B), 1)
        P = bb[:, None] == iota
        q = q_scr[...]
        S = lax.dot_general(h2b, q, (((1,), (1,)), ((), ())),
                            preferred_element_type=jnp.float32)
        Sm = jnp.where(P, S, _NEG)
        colmax = jnp.max(Sm, axis=0, keepdims=True)
        m_old = m_scr[...]
        m_new = jnp.maximum(m_old, colmax)
        rescale = jnp.exp(m_old - m_new)
        At = jnp.where(P, jnp.exp(S - m_new), 0.0)
        sacc_scr[...] = sacc_scr[...] * rescale + jnp.sum(At, axis=0,
                                                          keepdims=True)
        rnum_scr[...] = (rnum_scr[...] * rescale
                         + lax.dot_general(h2b, At, (((0,), (0,)), ((), ())),
                                           preferred_element_type=jnp.float32))
        m_scr[...] = m_new

    @pl.when((j == _NB + 1) & (st == _STEPS - 1))
    def head_phase():
        r_t = rnum_scr[...] / (sacc_scr[...] + 1e-16)
        q_star = jnp.concatenate([q_scr[...], r_t.T], axis=-1)
        o1 = jax.nn.relu(
            jnp.dot(q_star, w1_ref[...], preferred_element_type=jnp.float32)
            + b1_ref[...])
        out_ref[...] = (jnp.dot(o1, w2_ref[...],
                                preferred_element_type=jnp.float32)
                        + b2_ref[...])


def _set2set(h2, batch3, Wi, Wh, bl, W1, b1, W2, b2):
    grid = (_STEPS, _NB + 2)

    def nb_map(st, j):
        jj = jnp.clip(j - 1, 0, _NB - 1)
        return (jj, 0)

    return pl.pallas_call(
        _set2set_body,
        grid=grid,
        in_specs=[
            pl.BlockSpec((_TN2, DIM), nb_map),
            pl.BlockSpec((1, 1, _TN2), lambda st, j: (jnp.clip(j - 1, 0, _NB - 1), 0, 0)),
            pl.BlockSpec((2 * DIM, 4 * DIM), lambda st, j: (0, 0)),
            pl.BlockSpec((DIM, 4 * DIM), lambda st, j: (0, 0)),
            pl.BlockSpec((1, 4 * DIM), lambda st, j: (0, 0)),
            pl.BlockSpec((2 * DIM, DIM), lambda st, j: (0, 0)),
            pl.BlockSpec((1, DIM), lambda st, j: (0, 0)),
            pl.BlockSpec((DIM, 1), lambda st, j: (0, 0)),
            pl.BlockSpec((1, 1), lambda st, j: (0, 0)),
        ],
        out_specs=pl.BlockSpec((B, 1), lambda st, j: (0, 0)),
        out_shape=jax.ShapeDtypeStruct((B, 1), jnp.float32),
        scratch_shapes=[
            pltpu.VMEM((B, DIM), jnp.float32),
            pltpu.VMEM((B, DIM), jnp.float32),
            pltpu.VMEM((B, DIM), jnp.float32),
            pltpu.VMEM((1, B), jnp.float32),
            pltpu.VMEM((1, B), jnp.float32),
            pltpu.VMEM((DIM, B), jnp.float32),
        ],
    )(h2, batch3, Wi, Wh, bl.reshape(1, 4 * DIM), W1, b1.reshape(1, DIM),
      W2, b2.reshape(1, 1))


def kernel(x, edge_index, edge_attr, batch, W0, b0, Wm, We, bconv, Wroot,
           Wi, Wh, bl, W1, b1, W2, b2):
    src = edge_index[0].astype(jnp.int32)
    dst = edge_index[1].astype(jnp.int32)

    pe = E_PAD - E
    src_r = jnp.concatenate([src, jnp.zeros((pe,), jnp.int32)])
    dst_r = jnp.concatenate([dst, jnp.full((pe,), DUMMY, jnp.int32)])

    g2, hr = _node_prework(x, W0, b0, Wm, Wroot)
    ea2 = _edge_prework(edge_attr, We, bconv)

    g_flat = g2.reshape(2 * N, H)
    ea_flat = ea2.reshape(2 * EQ, 128)
    aggr_flat = _sc_edge_aggregate(g_flat, ea_flat, src_r, dst_r)
    aggr = aggr_flat.reshape(2, NPAD, H)[:, :N, :]

    h2 = _h2_assemble(hr, aggr)

    batch3 = batch.astype(jnp.int32).reshape(_NB, 1, _TN2)
    out = _set2set(h2, batch3, Wi, Wh, bl, W1, b1, W2, b2)
    return out.reshape(-1)


# speedup vs baseline: 1.1031x; 1.1031x over previous
import functools

import jax
import jax.numpy as jnp
from jax import lax
from jax.experimental import pallas as pl
from jax.experimental.pallas import tpu as pltpu
from jax.experimental.pallas import tpu_sc as plsc

N = 50000
E = 800000
DIN = 25
DIM = 64
FAB = 16
B = 512
H = DIM // 2

NC = 2
NS = 16
L = 16


_TN1 = 1000


def _node_prework_body(x_ref, w0_ref, b0_ref, wm_ref, wroot_ref, g_ref, hr_ref):
    h = jax.nn.relu(jnp.dot(x_ref[...], w0_ref[...],
                            preferred_element_type=jnp.float32) + b0_ref[...])
    g = jnp.dot(h, wm_ref[...], preferred_element_type=jnp.float32)
    g_ref[0] = g[:, :H]
    g_ref[1] = g[:, H:]
    hr_ref[...] = jnp.dot(h, wroot_ref[...], preferred_element_type=jnp.float32)


def _node_prework(x, W0, b0, Wm, Wroot):
    grid = (N // _TN1,)
    return pl.pallas_call(
        _node_prework_body,
        grid=grid,
        in_specs=[
            pl.BlockSpec((_TN1, DIN), lambda i: (i, 0)),
            pl.BlockSpec((DIN, DIM), lambda i: (0, 0)),
            pl.BlockSpec((1, DIM), lambda i: (0, 0)),
            pl.BlockSpec((DIM, DIM), lambda i: (0, 0)),
            pl.BlockSpec((DIM, DIM), lambda i: (0, 0)),
        ],
        out_specs=[
            pl.BlockSpec((2, _TN1, H), lambda i: (0, i, 0)),
            pl.BlockSpec((_TN1, DIM), lambda i: (i, 0)),
        ],
        out_shape=[
            jax.ShapeDtypeStruct((2, N, H), jnp.float32),
            jax.ShapeDtypeStruct((N, DIM), jnp.float32),
        ],
    )(x, W0, b0.reshape(1, DIM), Wm, Wroot)


E_PAD = 802816
EQ = E_PAD // 4
_TE4 = 2048


def _edge_prework_body(attr4_ref, wbd_ref, bc_ref, ea_ref):
    a = attr4_ref[...]
    ea_ref[0] = jnp.dot(a, wbd_ref[0],
                        preferred_element_type=jnp.float32) + bc_ref[0]
    ea_ref[1] = jnp.dot(a, wbd_ref[1],
                        preferred_element_type=jnp.float32) + bc_ref[1]


def _edge_prework(attr4, Wbd, bc2):
    grid = (EQ // _TE4,)
    return pl.pallas_call(
        _edge_prework_body,
        grid=grid,
        in_specs=[
            pl.BlockSpec((_TE4, 4 * FAB), lambda i: (i, 0)),
            pl.BlockSpec((2, 4 * FAB, 128), lambda i: (0, 0, 0)),
            pl.BlockSpec((2, 1, 128), lambda i: (0, 0, 0)),
        ],
        out_specs=pl.BlockSpec((2, _TE4, 128), lambda i: (0, i, 0)),
        out_shape=jax.ShapeDtypeStruct((2, EQ, 128), jnp.float32),
    )(attr4, Wbd, bc2)


EPT = E_PAD // NS
CH = 128
ERC = CH // 4
NCHK = EPT // CH
NPAIR = (NCHK - 2) // 2
NPAD = 50176
RPT = NPAD // NS
ZR = 56
DUMMY = N


def _sc_edge_body(g_hbm, ea_hbm, src_hbm, dst_hbm, out_hbm,
                  src0, dst0, ea0, gr0, dsc0, src1, dst1, ea1, gr1, dsc1,
                  zbuf, aggr_sh, si0, sg0, ss0, si1, sg1, ss1):
    c = lax.axis_index("c")
    s = lax.axis_index("s")
    zvec = jnp.zeros((L,), jnp.float32)

    def zrow(r, _):
        zbuf[r, pl.ds(0, L)] = zvec
        zbuf[r, pl.ds(L, L)] = zvec
        return 0
    lax.fori_loop(0, ZR, zrow, 0)
    def zcopy(z, _):
        pltpu.sync_copy(zbuf, aggr_sh.at[pl.ds(s * RPT + z * ZR, ZR)])
        return 0
    lax.fori_loop(0, RPT // ZR, zcopy, 0)
    plsc.subcore_barrier()

    ebase = s * EPT
    arow0 = c * EQ + s * (EPT // 4)
    goff = c * N

    bufs = ((src0, dst0, ea0, gr0, dsc0, si0, sg0, ss0),
            (src1, dst1, ea1, gr1, dsc1, si1, sg1, ss1))

    def fire_in(j, u):
        sv, dv, ev, gv, dsc, si, sg, ss = bufs[u]
        jf = jnp.minimum(j, NCHK - 1)
        pltpu.async_copy(src_hbm.at[pl.ds(ebase + jf * CH, CH)], sv, si)
        pltpu.async_copy(dst_hbm.at[pl.ds(ebase + jf * CH, CH)], dv, si)
        pltpu.async_copy(ea_hbm.at[pl.ds(arow0 + jf * ERC, ERC)], ev, si)

    def wait_in(u):
        sv, dv, ev, gv, dsc, si, sg, ss = bufs[u]
        pltpu.make_async_copy(src_hbm.at[pl.ds(ebase, CH)], sv, si).wait()
        pltpu.make_async_copy(dst_hbm.at[pl.ds(ebase, CH)], dv, si).wait()
        pltpu.make_async_copy(ea_hbm.at[pl.ds(arow0, ERC)], ev, si).wait()

    def add_off(u):
        sv = bufs[u][0]
        for w in range(CH // L):
            sv[pl.ds(w * L, L)] = sv[pl.ds(w * L, L)] + goff

    def fire_gather(u):
        sv, dv, ev, gv, dsc, si, sg, ss = bufs[u]
        return pltpu.async_copy(g_hbm.at[sv], gv, sg)

    def wait_gather(u):
        sv, dv, ev, gv, dsc, si, sg, ss = bufs[u]
        pltpu.make_async_copy(g_hbm.at[sv], gv, sg).wait()

    def compute(u):
        sv, dv, ev, gv, dsc, si, sg, ss = bufs[u]
        def body(k, _):
            for m in range(4):
                r = k * 4 + m
                for hh in range(2):
                    w = 2 * m + hh
                    val = gv[r, pl.ds(hh * L, L)] + ev[k, pl.ds(w * L, L)]
                    gv[r, pl.ds(hh * L, L)] = jnp.maximum(val, 0.0)
            return 0
        lax.fori_loop(0, CH // 4, body, 0)

    def copy_dst(u):
        dv, dsc = bufs[u][1], bufs[u][4]
        for w in range(CH // L):
            dsc[pl.ds(w * L, L)] = dv[pl.ds(w * L, L)]

    def fire_scatter(u):
        sv, dv, ev, gv, dsc, si, sg, ss = bufs[u]
        pltpu.async_copy(gv, aggr_sh.at[dsc], ss, add=True)

    def wait_scatter(u):
        sv, dv, ev, gv, dsc, si, sg, ss = bufs[u]
        pltpu.make_async_copy(gv, aggr_sh.at[dsc], ss).wait()

    fire_in(0, 0)
    fire_in(1, 1)
    wait_in(0)
    add_off(0)
    fire_gather(0).wait()
    compute(0)
    copy_dst(0)
    fire_scatter(0)
    fire_in(2, 0)
    wait_in(1)
    add_off(1)
    fire_gather(1)

    def iter_chunk(j, u):
        wait_scatter(1 - u)
        wait_in(1 - u)
        add_off(1 - u)
        fire_gather(1 - u)
        wait_gather(u)
        compute(u)
        copy_dst(u)
        fire_scatter(u)
        fire_in(j + 2, u)

    def do_pair(t, _):
        j0 = 1 + 2 * t
        iter_chunk(j0, 1)
        iter_chunk(j0 + 1, 0)
        return 0
    lax.fori_loop(0, NPAIR, do_pair, 0)

    wait_scatter(0)
    wait_gather(1)
    compute(1)
    copy_dst(1)
    sv, dv, ev, gv, dsc, si, sg, ss = bufs[1]
    pltpu.sync_copy(gv, aggr_sh.at[dsc], add=True)
    wait_in(0)

    plsc.subcore_barrier()
    pltpu.sync_copy(aggr_sh.at[pl.ds(s * RPT, RPT)],
                    out_hbm.at[pl.ds(c * NPAD + s * RPT, RPT)])


def _sc_edge_aggregate(g_flat, ea_flat, src, dst):
    mesh = plsc.VectorSubcoreMesh(core_axis_name="c", subcore_axis_name="s")
    return pl.kernel(
        _sc_edge_body,
        out_type=jax.ShapeDtypeStruct((2 * NPAD, H), jnp.float32),
        mesh=mesh,
        compiler_params=pltpu.CompilerParams(use_tc_tiling_on_sc=False),
        scratch_types=[
            pltpu.VMEM((CH,), jnp.int32),
            pltpu.VMEM((CH,), jnp.int32),
            pltpu.VMEM((ERC, 128), jnp.float32),
            pltpu.VMEM((CH, H), jnp.float32),
            pltpu.VMEM((CH,), jnp.int32),
            pltpu.VMEM((CH,), jnp.int32),
            pltpu.VMEM((CH,), jnp.int32),
            pltpu.VMEM((ERC, 128), jnp.float32),
            pltpu.VMEM((CH, H), jnp.float32),
            pltpu.VMEM((CH,), jnp.int32),
            pltpu.VMEM((ZR, H), jnp.float32),
            pltpu.VMEM_SHARED((NPAD, H), jnp.float32),
            pltpu.SemaphoreType.DMA,
            pltpu.SemaphoreType.DMA,
            pltpu.SemaphoreType.DMA,
            pltpu.SemaphoreType.DMA,
            pltpu.SemaphoreType.DMA,
            pltpu.SemaphoreType.DMA,
        ],
    )(g_flat, ea_flat, src, dst)


def _h2_body(hr_ref, a_ref, h2_ref):
    a = jnp.concatenate([a_ref[0], a_ref[1]], axis=-1)
    h2_ref[...] = jax.nn.relu(hr_ref[...] + a)


def _h2_assemble(hr, aggr):
    grid = (N // _TN1,)
    return pl.pallas_call(
        _h2_body,
        grid=grid,
        in_specs=[
            pl.BlockSpec((_TN1, DIM), lambda i: (i, 0)),
            pl.BlockSpec((2, _TN1, H), lambda i: (0, i, 0)),
        ],
        out_specs=pl.BlockSpec((_TN1, DIM), lambda i: (i, 0)),
        out_shape=jax.ShapeDtypeStruct((N, DIM), jnp.float32),
    )(hr, aggr)


_TN2 = 2000
_NB = N // _TN2
_STEPS = 3
_NEG = -1e30


def _set2set_body(h2_ref, batch_ref, wi_ref, wh_ref, bl_ref,
                  w1_ref, b1_ref, w2_ref, b2_ref, out_ref,
                  hs_scr, cs_scr, q_scr, m_scr, sacc_scr, rnum_scr):
    st = pl.program_id(0)
    j = pl.program_id(1)

    @pl.when(j == 0)
    def lstm_phase():
        is0 = (st == 0).astype(jnp.float32)
        r_t = rnum_scr[...] / (sacc_scr[...] + 1e-16)
        r = r_t.T
        q_prev = q_scr[...]
        keep = 1.0 - is0
        q_star = jnp.concatenate([q_prev, r], axis=-1)
        q_star = jnp.where(st == 0, 0.0, q_star)
        hs_prev = jnp.where(st == 0, 0.0, hs_scr[...])
        cs_prev = jnp.where(st == 0, 0.0, cs_scr[...])
        del keep
        z = (jnp.dot(q_star, wi_ref[...], preferred_element_type=jnp.float32)
             + jnp.dot(hs_prev, wh_ref[...], preferred_element_type=jnp.float32)
             + bl_ref[...])
        i_g = jax.nn.sigmoid(z[:, 0 * DIM:1 * DIM])
        f_g = jax.nn.sigmoid(z[:, 1 * DIM:2 * DIM])
        g_g = jnp.tanh(z[:, 2 * DIM:3 * DIM])
        o_g = jax.nn.sigmoid(z[:, 3 * DIM:4 * DIM])
        cs_new = f_g * cs_prev + i_g * g_g
        hs_new = o_g * jnp.tanh(cs_new)
        hs_scr[...] = hs_new
        cs_scr[...] = cs_new
        q_scr[...] = hs_new
        m_scr[...] = jnp.full((1, B), _NEG, jnp.float32)
        sacc_scr[...] = jnp.zeros((1, B), jnp.float32)
        rnum_scr[...] = jnp.zeros((DIM, B), jnp.float32)

    @pl.when((j >= 1) & (j <= _NB))
    def node_phase():
        h2b = h2_ref[...]
        bb = batch_ref[0, 0, :]
        iota = lax.broadcasted_iota(jnp.int32, (1, B), 1)
        P = bb[:, None] == iota
        q = q_scr[...]
        S = lax.dot_general(h2b, q, (((1,), (1,)), ((), ())),
                            preferred_element_type=jnp.float32)
        Sm = jnp.where(P, S, _NEG)
        colmax = jnp.max(Sm, axis=0, keepdims=True)
        m_old = m_scr[...]
        m_new = jnp.maximum(m_old, colmax)
        rescale = jnp.exp(m_old - m_new)
        At = jnp.where(P, jnp.exp(S - m_new), 0.0)
        sacc_scr[...] = sacc_scr[...] * rescale + jnp.sum(At, axis=0,
                                                          keepdims=True)
        rnum_scr[...] = (rnum_scr[...] * rescale
                         + lax.dot_general(h2b, At, (((0,), (0,)), ((), ())),
                                           preferred_element_type=jnp.float32))
        m_scr[...] = m_new

    @pl.when((j == _NB + 1) & (st == _STEPS - 1))
    def head_phase():
        r_t = rnum_scr[...] / (sacc_scr[...] + 1e-16)
        q_star = jnp.concatenate([q_scr[...], r_t.T], axis=-1)
        o1 = jax.nn.relu(
            jnp.dot(q_star, w1_ref[...], preferred_element_type=jnp.float32)
            + b1_ref[...])
        out_ref[...] = (jnp.dot(o1, w2_ref[...],
                                preferred_element_type=jnp.float32)
                        + b2_ref[...])


def _set2set(h2, batch3, Wi, Wh, bl, W1, b1, W2, b2):
    grid = (_STEPS, _NB + 2)

    def nb_map(st, j):
        jj = jnp.clip(j - 1, 0, _NB - 1)
        return (jj, 0)

    return pl.pallas_call(
        _set2set_body,
        grid=grid,
        in_specs=[
            pl.BlockSpec((_TN2, DIM), nb_map),
            pl.BlockSpec((1, 1, _TN2), lambda st, j: (jnp.clip(j - 1, 0, _NB - 1), 0, 0)),
            pl.BlockSpec((2 * DIM, 4 * DIM), lambda st, j: (0, 0)),
            pl.BlockSpec((DIM, 4 * DIM), lambda st, j: (0, 0)),
            pl.BlockSpec((1, 4 * DIM), lambda st, j: (0, 0)),
            pl.BlockSpec((2 * DIM, DIM), lambda st, j: (0, 0)),
            pl.BlockSpec((1, DIM), lambda st, j: (0, 0)),
            pl.BlockSpec((DIM, 1), lambda st, j: (0, 0)),
            pl.BlockSpec((1, 1), lambda st, j: (0, 0)),
        ],
        out_specs=pl.BlockSpec((B, 1), lambda st, j: (0, 0)),
        out_shape=jax.ShapeDtypeStruct((B, 1), jnp.float32),
        scratch_shapes=[
            pltpu.VMEM((B, DIM), jnp.float32),
            pltpu.VMEM((B, DIM), jnp.float32),
            pltpu.VMEM((B, DIM), jnp.float32),
            pltpu.VMEM((1, B), jnp.float32),
            pltpu.VMEM((1, B), jnp.float32),
            pltpu.VMEM((DIM, B), jnp.float32),
        ],
    )(h2, batch3, Wi, Wh, bl.reshape(1, 4 * DIM), W1, b1.reshape(1, DIM),
      W2, b2.reshape(1, 1))


def kernel(x, edge_index, edge_attr, batch, W0, b0, Wm, We, bconv, Wroot,
           Wi, Wh, bl, W1, b1, W2, b2):
    src = edge_index[0].astype(jnp.int32)
    dst = edge_index[1].astype(jnp.int32)

    pe = E_PAD - E
    src_p = jnp.concatenate([src, jnp.zeros((pe,), jnp.int32)])
    dst_p = jnp.concatenate([dst, jnp.full((pe,), DUMMY, jnp.int32)])
    attr_p = jnp.concatenate(
        [edge_attr, jnp.zeros((pe, FAB), jnp.float32)], axis=0)
    attr4 = attr_p.reshape(EQ, 4 * FAB)

    Wbd = jnp.stack([jnp.kron(jnp.eye(4, dtype=jnp.float32), We[:, :H]),
                     jnp.kron(jnp.eye(4, dtype=jnp.float32), We[:, H:])])
    bc2 = jnp.stack([jnp.tile(bconv[:H], 4),
                     jnp.tile(bconv[H:], 4)]).reshape(2, 1, 128)

    g2, hr = _node_prework(x, W0, b0, Wm, Wroot)
    ea2 = _edge_prework(attr4, Wbd, bc2)

    g_flat = g2.reshape(2 * N, H)
    ea_flat = ea2.reshape(2 * EQ, 128)
    aggr_flat = _sc_edge_aggregate(g_flat, ea_flat, src_p, dst_p)
    aggr = aggr_flat.reshape(2, NPAD, H)[:, :N, :]

    h2 = _h2_assemble(hr, aggr)

    batch3 = batch.astype(jnp.int32).reshape(_NB, 1, _TN2)
    out = _set2set(h2, batch3, Wi, Wh, bl, W1, b1, W2, b2)
    return out.reshape(-1)


# speedup vs baseline: 1.3188x; 1.1955x over previous
import functools

import jax
import jax.numpy as jnp
from jax import lax
from jax.experimental import pallas as pl
from jax.experimental.pallas import tpu as pltpu
from jax.experimental.pallas import tpu_sc as plsc

N = 50000
E = 800000
DIN = 25
DIM = 64
FAB = 16
B = 512
H = DIM // 2

NC = 2
NS = 16
L = 16


_TN1 = 1000


def _node_prework_body(x_ref, w0_ref, b0_ref, wm_ref, wroot_ref, g_ref, hr_ref):
    h = jax.nn.relu(jnp.dot(x_ref[...], w0_ref[...],
                            preferred_element_type=jnp.float32) + b0_ref[...])
    g = jnp.dot(h, wm_ref[...], preferred_element_type=jnp.float32)
    g_ref[0] = g[:, :H]
    g_ref[1] = g[:, H:]
    hr_ref[...] = jnp.dot(h, wroot_ref[...], preferred_element_type=jnp.float32)


def _node_prework(x, W0, b0, Wm, Wroot):
    grid = (N // _TN1,)
    return pl.pallas_call(
        _node_prework_body,
        grid=grid,
        in_specs=[
            pl.BlockSpec((_TN1, DIN), lambda i: (i, 0)),
            pl.BlockSpec((DIN, DIM), lambda i: (0, 0)),
            pl.BlockSpec((1, DIM), lambda i: (0, 0)),
            pl.BlockSpec((DIM, DIM), lambda i: (0, 0)),
            pl.BlockSpec((DIM, DIM), lambda i: (0, 0)),
        ],
        out_specs=[
            pl.BlockSpec((2, _TN1, H), lambda i: (0, i, 0)),
            pl.BlockSpec((_TN1, DIM), lambda i: (i, 0)),
        ],
        out_shape=[
            jax.ShapeDtypeStruct((2, N, H), jnp.float32),
            jax.ShapeDtypeStruct((N, DIM), jnp.float32),
        ],
    )(x, W0, b0.reshape(1, DIM), Wm, Wroot)


E_PAD = 802816
EQ = E_PAD // 4
_TE4 = 2048


def _edge_prework_body(attr4_ref, wbd_ref, bc_ref, ea_ref):
    a = attr4_ref[...]
    ea_ref[0] = jnp.dot(a, wbd_ref[0],
                        preferred_element_type=jnp.float32) + bc_ref[0]
    ea_ref[1] = jnp.dot(a, wbd_ref[1],
                        preferred_element_type=jnp.float32) + bc_ref[1]


def _edge_prework(attr4, Wbd, bc2):
    grid = (EQ // _TE4,)
    return pl.pallas_call(
        _edge_prework_body,
        grid=grid,
        in_specs=[
            pl.BlockSpec((_TE4, 4 * FAB),
                         lambda i: (jnp.minimum(i, E // 4 // _TE4), 0)),
            pl.BlockSpec((2, 4 * FAB, 128), lambda i: (0, 0, 0)),
            pl.BlockSpec((2, 1, 128), lambda i: (0, 0, 0)),
        ],
        out_specs=pl.BlockSpec((2, _TE4, 128), lambda i: (0, i, 0)),
        out_shape=jax.ShapeDtypeStruct((2, EQ, 128), jnp.float32),
    )(attr4, Wbd, bc2)


EPT = E_PAD // NS
CH = 128
ERC = CH // 4
NCHK = EPT // CH
NPAIR = (NCHK - 2) // 2
NPAD = 50176
RPT = NPAD // NS
ZR = 56
DUMMY = N


def _sc_edge_body(g_hbm, ea_hbm, src_hbm, dst_hbm, out_hbm,
                  src0, dst0, ea0, gr0, dsc0, src1, dst1, ea1, gr1, dsc1,
                  zbuf, aggr_sh, si0, sg0, ss0, si1, sg1, ss1):
    c = lax.axis_index("c")
    s = lax.axis_index("s")
    zvec = jnp.zeros((L,), jnp.float32)

    def zrow(r, _):
        zbuf[r, pl.ds(0, L)] = zvec
        zbuf[r, pl.ds(L, L)] = zvec
        return 0
    lax.fori_loop(0, ZR, zrow, 0)
    def zcopy(z, _):
        pltpu.sync_copy(zbuf, aggr_sh.at[pl.ds(s * RPT + z * ZR, ZR)])
        return 0
    lax.fori_loop(0, RPT // ZR, zcopy, 0)
    plsc.subcore_barrier()

    ebase = s * EPT
    arow0 = c * EQ + s * (EPT // 4)
    goff = c * N

    bufs = ((src0, dst0, ea0, gr0, dsc0, si0, sg0, ss0),
            (src1, dst1, ea1, gr1, dsc1, si1, sg1, ss1))

    def fire_in(j, u):
        sv, dv, ev, gv, dsc, si, sg, ss = bufs[u]
        jf = jnp.minimum(j, NCHK - 1)
        pltpu.async_copy(src_hbm.at[pl.ds(ebase + jf * CH, CH)], sv, si)
        pltpu.async_copy(dst_hbm.at[pl.ds(ebase + jf * CH, CH)], dv, si)
        pltpu.async_copy(ea_hbm.at[pl.ds(arow0 + jf * ERC, ERC)], ev, si)

    def wait_in(u):
        sv, dv, ev, gv, dsc, si, sg, ss = bufs[u]
        pltpu.make_async_copy(src_hbm.at[pl.ds(ebase, CH)], sv, si).wait()
        pltpu.make_async_copy(dst_hbm.at[pl.ds(ebase, CH)], dv, si).wait()
        pltpu.make_async_copy(ea_hbm.at[pl.ds(arow0, ERC)], ev, si).wait()

    def add_off(u):
        sv = bufs[u][0]
        for w in range(CH // L):
            sv[pl.ds(w * L, L)] = sv[pl.ds(w * L, L)] + goff

    def fire_gather(u):
        sv, dv, ev, gv, dsc, si, sg, ss = bufs[u]
        return pltpu.async_copy(g_hbm.at[sv], gv, sg)

    def wait_gather(u):
        sv, dv, ev, gv, dsc, si, sg, ss = bufs[u]
        pltpu.make_async_copy(g_hbm.at[sv], gv, sg).wait()

    def compute(u):
        sv, dv, ev, gv, dsc, si, sg, ss = bufs[u]
        def body(k, _):
            for m in range(4):
                r = k * 4 + m
                for hh in range(2):
                    w = 2 * m + hh
                    val = gv[r, pl.ds(hh * L, L)] + ev[k, pl.ds(w * L, L)]
                    gv[r, pl.ds(hh * L, L)] = jnp.maximum(val, 0.0)
            return 0
        lax.fori_loop(0, CH // 4, body, 0)

    def copy_dst(u):
        dv, dsc = bufs[u][1], bufs[u][4]
        for w in range(CH // L):
            dsc[pl.ds(w * L, L)] = dv[pl.ds(w * L, L)]

    def fire_scatter(u):
        sv, dv, ev, gv, dsc, si, sg, ss = bufs[u]
        pltpu.async_copy(gv, aggr_sh.at[dsc], ss, add=True)

    def wait_scatter(u):
        sv, dv, ev, gv, dsc, si, sg, ss = bufs[u]
        pltpu.make_async_copy(gv, aggr_sh.at[dsc], ss).wait()

    fire_in(0, 0)
    fire_in(1, 1)
    wait_in(0)
    add_off(0)
    fire_gather(0).wait()
    compute(0)
    copy_dst(0)
    fire_scatter(0)
    fire_in(2, 0)
    wait_in(1)
    add_off(1)
    fire_gather(1)

    def iter_chunk(j, u):
        wait_scatter(1 - u)
        wait_in(1 - u)
        add_off(1 - u)
        fire_gather(1 - u)
        wait_gather(u)
        compute(u)
        copy_dst(u)
        fire_scatter(u)
        fire_in(j + 2, u)

    def do_pair(t, _):
        j0 = 1 + 2 * t
        iter_chunk(j0, 1)
        iter_chunk(j0 + 1, 0)
        return 0
    lax.fori_loop(0, NPAIR, do_pair, 0)

    wait_scatter(0)
    wait_gather(1)
    compute(1)
    copy_dst(1)
    sv, dv, ev, gv, dsc, si, sg, ss = bufs[1]
    pltpu.sync_copy(gv, aggr_sh.at[dsc], add=True)
    wait_in(0)

    plsc.subcore_barrier()
    pltpu.sync_copy(aggr_sh.at[pl.ds(s * RPT, RPT)],
                    out_hbm.at[pl.ds(c * NPAD + s * RPT, RPT)])


def _sc_edge_aggregate(g_flat, ea_flat, src, dst):
    mesh = plsc.VectorSubcoreMesh(core_axis_name="c", subcore_axis_name="s")
    return pl.kernel(
        _sc_edge_body,
        out_type=jax.ShapeDtypeStruct((2 * NPAD, H), jnp.float32),
        mesh=mesh,
        compiler_params=pltpu.CompilerParams(use_tc_tiling_on_sc=False),
        scratch_types=[
            pltpu.VMEM((CH,), jnp.int32),
            pltpu.VMEM((CH,), jnp.int32),
            pltpu.VMEM((ERC, 128), jnp.float32),
            pltpu.VMEM((CH, H), jnp.float32),
            pltpu.VMEM((CH,), jnp.int32),
            pltpu.VMEM((CH,), jnp.int32),
            pltpu.VMEM((CH,), jnp.int32),
            pltpu.VMEM((ERC, 128), jnp.float32),
            pltpu.VMEM((CH, H), jnp.float32),
            pltpu.VMEM((CH,), jnp.int32),
            pltpu.VMEM((ZR, H), jnp.float32),
            pltpu.VMEM_SHARED((NPAD, H), jnp.float32),
            pltpu.SemaphoreType.DMA,
            pltpu.SemaphoreType.DMA,
            pltpu.SemaphoreType.DMA,
            pltpu.SemaphoreType.DMA,
            pltpu.SemaphoreType.DMA,
            pltpu.SemaphoreType.DMA,
        ],
    )(g_flat, ea_flat, src, dst)


def _h2_body(hr_ref, a_ref, h2_ref):
    a = jnp.concatenate([a_ref[0], a_ref[1]], axis=-1)
    h2_ref[...] = jax.nn.relu(hr_ref[...] + a)


def _h2_assemble(hr, aggr):
    grid = (N // _TN1,)
    return pl.pallas_call(
        _h2_body,
        grid=grid,
        in_specs=[
            pl.BlockSpec((_TN1, DIM), lambda i: (i, 0)),
            pl.BlockSpec((2, _TN1, H), lambda i: (0, i, 0)),
        ],
        out_specs=pl.BlockSpec((_TN1, DIM), lambda i: (i, 0)),
        out_shape=jax.ShapeDtypeStruct((N, DIM), jnp.float32),
    )(hr, aggr)


_TN2 = 2000
_NB = N // _TN2
_STEPS = 3
_NEG = -1e30


def _set2set_body(h2_ref, batch_ref, wi_ref, wh_ref, bl_ref,
                  w1_ref, b1_ref, w2_ref, b2_ref, out_ref,
                  hs_scr, cs_scr, q_scr, m_scr, sacc_scr, rnum_scr):
    st = pl.program_id(0)
    j = pl.program_id(1)

    @pl.when(j == 0)
    def lstm_phase():
        is0 = (st == 0).astype(jnp.float32)
        r_t = rnum_scr[...] / (sacc_scr[...] + 1e-16)
        r = r_t.T
        q_prev = q_scr[...]
        keep = 1.0 - is0
        q_star = jnp.concatenate([q_prev, r], axis=-1)
        q_star = jnp.where(st == 0, 0.0, q_star)
        hs_prev = jnp.where(st == 0, 0.0, hs_scr[...])
        cs_prev = jnp.where(st == 0, 0.0, cs_scr[...])
        del keep
        z = (jnp.dot(q_star, wi_ref[...], preferred_element_type=jnp.float32)
             + jnp.dot(hs_prev, wh_ref[...], preferred_element_type=jnp.float32)
             + bl_ref[...])
        i_g = jax.nn.sigmoid(z[:, 0 * DIM:1 * DIM])
        f_g = jax.nn.sigmoid(z[:, 1 * DIM:2 * DIM])
        g_g = jnp.tanh(z[:, 2 * DIM:3 * DIM])
        o_g = jax.nn.sigmoid(z[:, 3 * DIM:4 * DIM])
        cs_new = f_g * cs_prev + i_g * g_g
        hs_new = o_g * jnp.tanh(cs_new)
        hs_scr[...] = hs_new
        cs_scr[...] = cs_new
        q_scr[...] = hs_new
        m_scr[...] = jnp.full((1, B), _NEG, jnp.float32)
        sacc_scr[...] = jnp.zeros((1, B), jnp.float32)
        rnum_scr[...] = jnp.zeros((DIM, B), jnp.float32)

    @pl.when((j >= 1) & (j <= _NB))
    def node_phase():
        h2b = h2_ref[...]
        bb = batch_ref[0, 0, :]
        iota = lax.broadcasted_iota(jnp.int32, (1, B), 1)
        P = bb[:, None] == iota
        q = q_scr[...]
        S = lax.dot_general(h2b, q, (((1,), (1,)), ((), ())),
                            preferred_element_type=jnp.float32)
        Sm = jnp.where(P, S, _NEG)
        colmax = jnp.max(Sm, axis=0, keepdims=True)
        m_old = m_scr[...]
        m_new = jnp.maximum(m_old, colmax)
        rescale = jnp.exp(m_old - m_new)
        At = jnp.where(P, jnp.exp(S - m_new), 0.0)
        sacc_scr[...] = sacc_scr[...] * rescale + jnp.sum(At, axis=0,
                                                          keepdims=True)
        rnum_scr[...] = (rnum_scr[...] * rescale
                         + lax.dot_general(h2b, At, (((0,), (0,)), ((), ())),
                                           preferred_element_type=jnp.float32))
        m_scr[...] = m_new

    @pl.when((j == _NB + 1) & (st == _STEPS - 1))
    def head_phase():
        r_t = rnum_scr[...] / (sacc_scr[...] + 1e-16)
        q_star = jnp.concatenate([q_scr[...], r_t.T], axis=-1)
        o1 = jax.nn.relu(
            jnp.dot(q_star, w1_ref[...], preferred_element_type=jnp.float32)
            + b1_ref[...])
        out_ref[...] = (jnp.dot(o1, w2_ref[...],
                                preferred_element_type=jnp.float32)
                        + b2_ref[...])


def _set2set(h2, batch3, Wi, Wh, bl, W1, b1, W2, b2):
    grid = (_STEPS, _NB + 2)

    def nb_map(st, j):
        jj = jnp.clip(j - 1, 0, _NB - 1)
        return (jj, 0)

    return pl.pallas_call(
        _set2set_body,
        grid=grid,
        in_specs=[
            pl.BlockSpec((_TN2, DIM), nb_map),
            pl.BlockSpec((1, 1, _TN2), lambda st, j: (jnp.clip(j - 1, 0, _NB - 1), 0, 0)),
            pl.BlockSpec((2 * DIM, 4 * DIM), lambda st, j: (0, 0)),
            pl.BlockSpec((DIM, 4 * DIM), lambda st, j: (0, 0)),
            pl.BlockSpec((1, 4 * DIM), lambda st, j: (0, 0)),
            pl.BlockSpec((2 * DIM, DIM), lambda st, j: (0, 0)),
            pl.BlockSpec((1, DIM), lambda st, j: (0, 0)),
            pl.BlockSpec((DIM, 1), lambda st, j: (0, 0)),
            pl.BlockSpec((1, 1), lambda st, j: (0, 0)),
        ],
        out_specs=pl.BlockSpec((B, 1), lambda st, j: (0, 0)),
        out_shape=jax.ShapeDtypeStruct((B, 1), jnp.float32),
        scratch_shapes=[
            pltpu.VMEM((B, DIM), jnp.float32),
            pltpu.VMEM((B, DIM), jnp.float32),
            pltpu.VMEM((B, DIM), jnp.float32),
            pltpu.VMEM((1, B), jnp.float32),
            pltpu.VMEM((1, B), jnp.float32),
            pltpu.VMEM((DIM, B), jnp.float32),
        ],
    )(h2, batch3, Wi, Wh, bl.reshape(1, 4 * DIM), W1, b1.reshape(1, DIM),
      W2, b2.reshape(1, 1))


def kernel(x, edge_index, edge_attr, batch, W0, b0, Wm, We, bconv, Wroot,
           Wi, Wh, bl, W1, b1, W2, b2):
    src = edge_index[0].astype(jnp.int32)
    dst = edge_index[1].astype(jnp.int32)

    pe = E_PAD - E
    src_p = jnp.concatenate([src, jnp.zeros((pe,), jnp.int32)])
    dst_p = jnp.concatenate([dst, jnp.full((pe,), DUMMY, jnp.int32)])
    attr4 = edge_attr.reshape(E // 4, 4 * FAB)

    Wbd = jnp.stack([jnp.kron(jnp.eye(4, dtype=jnp.float32), We[:, :H]),
                     jnp.kron(jnp.eye(4, dtype=jnp.float32), We[:, H:])])
    bc2 = jnp.stack([jnp.tile(bconv[:H], 4),
                     jnp.tile(bconv[H:], 4)]).reshape(2, 1, 128)

    g2, hr = _node_prework(x, W0, b0, Wm, Wroot)
    ea2 = _edge_prework(attr4, Wbd, bc2)

    g_flat = g2.reshape(2 * N, H)
    ea_flat = ea2.reshape(2 * EQ, 128)
    aggr_flat = _sc_edge_aggregate(g_flat, ea_flat, src_p, dst_p)
    aggr = aggr_flat.reshape(2, NPAD, H)[:, :N, :]

    h2 = _h2_assemble(hr, aggr)

    batch3 = batch.astype(jnp.int32).reshape(_NB, 1, _TN2)
    out = _set2set(h2, batch3, Wi, Wh, bl, W1, b1, W2, b2)
    return out.reshape(-1)
